# baseline (speedup 1.0000x reference)
import functools

import jax
import jax.numpy as jnp
from jax import lax
from jax.experimental import pallas as pl
from jax.experimental.pallas import tpu as pltpu
from jax.experimental.pallas import tpu_sc as plsc

_DIL = 2
_NEG_INF = float("-inf")

_SC_INFO = plsc.get_sparse_core_info()
_NC = _SC_INFO.num_cores
_NS = _SC_INFO.num_subcores
_NW = _NC * _NS
_CHUNK = 128


def _dot(a, b, ca, cb):
    return lax.dot_general(a, b, (((ca,), (cb,)), ((), ())),
                           preferred_element_type=jnp.float32)


def _prep_body(mem_ref, wfb_ref, memB_ref):
    memB_ref[...] = _dot(mem_ref[...], wfb_ref[...], 1, 1)


def _score_body(x_ref, mem_ref, wfa_ref, yg_ref, idx_ref, w1_ref):
    P = x_ref.shape[2]
    M = mem_ref.shape[0]
    x = x_ref[0]
    mem = mem_ref[...]

    ig = jnp.mean(x, axis=1, keepdims=True)
    sg = _dot(mem, ig, 1, 0)
    sg = sg - jnp.max(sg, axis=0, keepdims=True)
    eg = jnp.exp(sg)
    smg = eg / jnp.sum(eg, axis=0, keepdims=True)
    mr = _dot(mem, smg, 0, 0) + ig
    gate = 1.0 / (1.0 + jnp.exp(-mr))
    yg_ref[0] = _dot(x * gate, wfa_ref[...], 0, 1)

    S = _dot(x, mem, 0, 1)
    col = lax.broadcasted_iota(jnp.int32, (P, M), 1)
    v1 = jnp.max(S, axis=1, keepdims=True)
    i1 = jnp.min(jnp.where(S == v1, col, M), axis=1, keepdims=True)
    S2 = jnp.where(col == i1, _NEG_INF, S)
    v2 = jnp.max(S2, axis=1, keepdims=True)
    i2 = jnp.min(jnp.where(S2 == v2, col, M), axis=1, keepdims=True)
    e2 = jnp.exp(v2 - v1)
    idx_ref[0, 0] = i1
    idx_ref[1, 0] = i2
    w1_ref[0] = 1.0 / (1.0 + e2)


def _make_sc_gather(n_idx, D):
    rows_per_w = n_idx // _NW
    chunks = rows_per_w // _CHUNK
    mesh = plsc.VectorSubcoreMesh(core_axis_name="c", subcore_axis_name="s")

    @functools.partial(
        pl.kernel, mesh=mesh,
        out_type=jax.ShapeDtypeStruct((n_idx, D), jnp.float32),
        scratch_types=[
            pltpu.VMEM((chunks, _CHUNK), jnp.int32),
            pltpu.VMEM((_CHUNK, D), jnp.float32),
            pltpu.SemaphoreType.DMA,
        ],
    )
    def gather_k(table_hbm, idx_hbm, out_hbm, idx_v, rows_v, sem):
        wid = lax.axis_index("s") * _NC + lax.axis_index("c")
        base_chunk = wid * chunks
        pltpu.sync_copy(idx_hbm.at[pl.ds(base_chunk, chunks)], idx_v)
        for j in range(chunks):
            pltpu.async_copy(table_hbm.at[idx_v.at[j]], rows_v, sem).wait()
            pltpu.sync_copy(
                rows_v, out_hbm.at[pl.ds((base_chunk + j) * _CHUNK, _CHUNK)])

    return gather_k


def _fuse_body(H, W, yg_ref, g1_ref, g2_ref, w1_ref, bf_ref, taps_ref,
               bdw_ref, out_ref):
    C = yg_ref.shape[2]
    a1 = w1_ref[0]
    Y = (yg_ref[0] + a1 * g1_ref[0, 0] + (1.0 - a1) * g2_ref[0, 0]
         + bf_ref[...])
    Y = jnp.where(Y > 0, Y, 0.2 * Y)

    Yh = Y.reshape(H, W, C)

    def shift(a, axis, d):
        if d == 0:
            return a
        zshape = list(a.shape)
        zshape[axis] = abs(d)
        z = jnp.zeros(zshape, a.dtype)
        n = a.shape[axis]
        if d > 0:
            body = lax.slice_in_dim(a, d, n, axis=axis)
            return jnp.concatenate([body, z], axis=axis)
        body = lax.slice_in_dim(a, 0, n + d, axis=axis)
        return jnp.concatenate([z, body], axis=axis)

    acc = jnp.zeros((H, W, C), jnp.float32)
    k = 0
    for kh in range(3):
        for kw in range(3):
            dh = (kh - 1) * _DIL
            dw = (kw - 1) * _DIL
            win = shift(shift(Yh, 0, dh), 1, dw)
            acc = acc + win * taps_ref[k, :][None, None, :]
            k += 1
    acc = acc + bdw_ref[0, :][None, None, :]
    out_ref[0] = jnp.where(acc > 0, acc, 0.2 * acc)


def kernel(image_feature, memory, W_fuse, b_fuse, W_dw, b_dw):
    B, C, H, W = image_feature.shape
    M = memory.shape[0]
    P = H * W
    GB = 4
    x_cp = image_feature.reshape(B, C, P)
    taps = W_dw[:, 0, :, :].reshape(C, 9).T
    bf = b_fuse.reshape(1, C)
    bdw = b_dw.reshape(1, C)

    memB = pl.pallas_call(
        _prep_body,
        grid=(1,),
        in_specs=[
            pl.BlockSpec((M, C), lambda i: (0, 0)),
            pl.BlockSpec((C, C), lambda i: (0, 1)),
        ],
        out_specs=pl.BlockSpec((M, C), lambda i: (0, 0)),
        out_shape=jax.ShapeDtypeStruct((M, C), jnp.float32),
    )(memory, W_fuse)

    sc_gather = _make_sc_gather(2 * GB * P, C)
    outs = []
    for g in range(B // GB):
        yg, idx, w1 = pl.pallas_call(
            _score_body,
            grid=(GB,),
            in_specs=[
                pl.BlockSpec((1, C, P), lambda b, g=g: (GB * g + b, 0, 0)),
                pl.BlockSpec((M, C), lambda b: (0, 0)),
                pl.BlockSpec((C, C), lambda b: (0, 0)),
            ],
            out_specs=[
                pl.BlockSpec((1, P, C), lambda b: (b, 0, 0)),
                pl.BlockSpec((2, 1, P, 1), lambda b: (0, b, 0, 0)),
                pl.BlockSpec((1, P, 1), lambda b: (b, 0, 0)),
            ],
            out_shape=[
                jax.ShapeDtypeStruct((GB, P, C), jnp.float32),
                jax.ShapeDtypeStruct((2, GB, P, 1), jnp.int32),
                jax.ShapeDtypeStruct((GB, P, 1), jnp.float32),
            ],
        )(x_cp, memory, W_fuse)

        n_idx = 2 * GB * P
        idx_flat = idx.reshape(n_idx // _CHUNK, _CHUNK)
        gathered = sc_gather(memB, idx_flat)
        gr = gathered.reshape(2, GB, P, C)

        outs.append(pl.pallas_call(
            functools.partial(_fuse_body, H, W),
            grid=(GB,),
            in_specs=[
                pl.BlockSpec((1, P, C), lambda b: (b, 0, 0)),
                pl.BlockSpec((1, 1, P, C), lambda b: (0, b, 0, 0)),
                pl.BlockSpec((1, 1, P, C), lambda b: (1, b, 0, 0)),
                pl.BlockSpec((1, P, 1), lambda b: (b, 0, 0)),
                pl.BlockSpec((1, C), lambda b: (0, 0)),
                pl.BlockSpec((9, C), lambda b: (0, 0)),
                pl.BlockSpec((1, C), lambda b: (0, 0)),
            ],
            out_specs=pl.BlockSpec((1, H, W, C), lambda b: (b, 0, 0, 0)),
            out_shape=jax.ShapeDtypeStruct((GB, H, W, C), jnp.float32),
        )(yg, gr, gr, w1, bf, taps, bdw))

    out = jnp.concatenate(outs, axis=0)
    return out


# speedup vs baseline: 1.0133x; 1.0133x over previous
import functools

import jax
import jax.numpy as jnp
from jax import lax
from jax.experimental import pallas as pl
from jax.experimental.pallas import tpu as pltpu
from jax.experimental.pallas import tpu_sc as plsc

_DIL = 2
_NEG_INF = float("-inf")

_SC_INFO = plsc.get_sparse_core_info()
_NC = _SC_INFO.num_cores
_NS = _SC_INFO.num_subcores
_NW = _NC * _NS
_CHUNK = 128


def _dot(a, b, ca, cb):
    return lax.dot_general(a, b, (((ca,), (cb,)), ((), ())),
                           preferred_element_type=jnp.float32)


def _prep_body(mem_ref, wfb_ref, memB_ref):
    C = mem_ref.shape[1]
    C2 = C // 2
    mb = _dot(mem_ref[...], wfb_ref[...], 1, 1).astype(jnp.bfloat16)
    lo = lax.bitcast_convert_type(mb[:, :C2], jnp.uint16).astype(jnp.uint32)
    hi = lax.bitcast_convert_type(mb[:, C2:], jnp.uint16).astype(jnp.uint32)
    packed = lo | (hi << 16)
    pad = memB_ref.shape[1] - C2
    memB_ref[...] = jnp.concatenate(
        [packed, jnp.zeros((packed.shape[0], pad), jnp.uint32)], axis=1)


def _score_body(x_ref, mem_ref, wfa_ref, yg_ref, idx_ref, w1_ref):
    P = x_ref.shape[2]
    M = mem_ref.shape[0]
    x = x_ref[0]
    mem = mem_ref[...]

    ig = jnp.mean(x, axis=1, keepdims=True)
    sg = _dot(mem, ig, 1, 0)
    sg = sg - jnp.max(sg, axis=0, keepdims=True)
    eg = jnp.exp(sg)
    smg = eg / jnp.sum(eg, axis=0, keepdims=True)
    mr = _dot(mem, smg, 0, 0) + ig
    gate = 1.0 / (1.0 + jnp.exp(-mr))
    yg_ref[0] = _dot(x * gate, wfa_ref[...], 0, 1)

    S = _dot(x, mem, 0, 1)
    col = lax.broadcasted_iota(jnp.int32, (P, M), 1)
    v1 = jnp.max(S, axis=1, keepdims=True)
    i1 = jnp.min(jnp.where(S == v1, col, M), axis=1, keepdims=True)
    S2 = jnp.where(col == i1, _NEG_INF, S)
    v2 = jnp.max(S2, axis=1, keepdims=True)
    i2 = jnp.min(jnp.where(S2 == v2, col, M), axis=1, keepdims=True)
    e2 = jnp.exp(v2 - v1)
    idx_ref[0, 0] = i1
    idx_ref[1, 0] = i2
    w1_ref[0] = 1.0 / (1.0 + e2)


def _make_sc_gather(n_idx, D):
    rows_per_w = n_idx // _NW
    chunks = rows_per_w // _CHUNK
    mesh = plsc.VectorSubcoreMesh(core_axis_name="c", subcore_axis_name="s")

    @functools.partial(
        pl.kernel, mesh=mesh,
        out_type=jax.ShapeDtypeStruct((n_idx, D), jnp.uint32),
        scratch_types=[
            pltpu.VMEM((chunks, _CHUNK), jnp.int32),
            pltpu.VMEM((_CHUNK, D), jnp.uint32),
            pltpu.SemaphoreType.DMA,
        ],
    )
    def gather_k(table_hbm, idx_hbm, out_hbm, idx_v, rows_v, sem):
        wid = lax.axis_index("s") * _NC + lax.axis_index("c")
        base_chunk = wid * chunks
        pltpu.sync_copy(idx_hbm.at[pl.ds(base_chunk, chunks)], idx_v)
        for j in range(chunks):
            pltpu.async_copy(table_hbm.at[idx_v.at[j]], rows_v, sem).wait()
            pltpu.sync_copy(
                rows_v, out_hbm.at[pl.ds((base_chunk + j) * _CHUNK, _CHUNK)])

    return gather_k


def _fuse_body(H, W, yg_ref, g1_ref, g2_ref, w1_ref, bf_ref, taps_ref,
               bdw_ref, out_ref):
    C = yg_ref.shape[2]
    a1 = w1_ref[0]

    def unpack(gu_padded):
        gu = gu_padded[:, :C // 2]
        lo = lax.bitcast_convert_type(
            (gu & 0xFFFF).astype(jnp.uint16), jnp.bfloat16).astype(jnp.float32)
        hi = lax.bitcast_convert_type(
            (gu >> 16).astype(jnp.uint16), jnp.bfloat16).astype(jnp.float32)
        return jnp.concatenate([lo, hi], axis=1)

    g1 = unpack(g1_ref[0, 0])
    g2 = unpack(g2_ref[0, 0])
    Y = (yg_ref[0] + a1 * g1 + (1.0 - a1) * g2
         + bf_ref[...])
    Y = jnp.where(Y > 0, Y, 0.2 * Y)

    Yh = Y.reshape(H, W, C)

    def shift(a, axis, d):
        if d == 0:
            return a
        zshape = list(a.shape)
        zshape[axis] = abs(d)
        z = jnp.zeros(zshape, a.dtype)
        n = a.shape[axis]
        if d > 0:
            body = lax.slice_in_dim(a, d, n, axis=axis)
            return jnp.concatenate([body, z], axis=axis)
        body = lax.slice_in_dim(a, 0, n + d, axis=axis)
        return jnp.concatenate([z, body], axis=axis)

    acc = jnp.zeros((H, W, C), jnp.float32)
    k = 0
    for kh in range(3):
        for kw in range(3):
            dh = (kh - 1) * _DIL
            dw = (kw - 1) * _DIL
            win = shift(shift(Yh, 0, dh), 1, dw)
            acc = acc + win * taps_ref[k, :][None, None, :]
            k += 1
    acc = acc + bdw_ref[0, :][None, None, :]
    out_ref[0] = jnp.where(acc > 0, acc, 0.2 * acc)


def kernel(image_feature, memory, W_fuse, b_fuse, W_dw, b_dw):
    B, C, H, W = image_feature.shape
    M = memory.shape[0]
    P = H * W
    GB = 4
    x_cp = image_feature.reshape(B, C, P)
    taps = W_dw[:, 0, :, :].reshape(C, 9).T
    bf = b_fuse.reshape(1, C)
    bdw = b_dw.reshape(1, C)

    PW = 256
    memB = pl.pallas_call(
        _prep_body,
        grid=(1,),
        in_specs=[
            pl.BlockSpec((M, C), lambda i: (0, 0)),
            pl.BlockSpec((C, C), lambda i: (0, 1)),
        ],
        out_specs=pl.BlockSpec((M, PW), lambda i: (0, 0)),
        out_shape=jax.ShapeDtypeStruct((M, PW), jnp.uint32),
    )(memory, W_fuse)

    sc_gather = _make_sc_gather(2 * GB * P, PW)
    outs = []
    for g in range(B // GB):
        yg, idx, w1 = pl.pallas_call(
            _score_body,
            grid=(GB,),
            in_specs=[
                pl.BlockSpec((1, C, P), lambda b, g=g: (GB * g + b, 0, 0)),
                pl.BlockSpec((M, C), lambda b: (0, 0)),
                pl.BlockSpec((C, C), lambda b: (0, 0)),
            ],
            out_specs=[
                pl.BlockSpec((1, P, C), lambda b: (b, 0, 0)),
                pl.BlockSpec((2, 1, P, 1), lambda b: (0, b, 0, 0)),
                pl.BlockSpec((1, P, 1), lambda b: (b, 0, 0)),
            ],
            out_shape=[
                jax.ShapeDtypeStruct((GB, P, C), jnp.float32),
                jax.ShapeDtypeStruct((2, GB, P, 1), jnp.int32),
                jax.ShapeDtypeStruct((GB, P, 1), jnp.float32),
            ],
        )(x_cp, memory, W_fuse)

        n_idx = 2 * GB * P
        idx_flat = idx.reshape(n_idx // _CHUNK, _CHUNK)
        gathered = sc_gather(memB, idx_flat)
        gr = gathered.reshape(2, GB, P, PW)

        outs.append(pl.pallas_call(
            functools.partial(_fuse_body, H, W),
            grid=(GB,),
            in_specs=[
                pl.BlockSpec((1, P, C), lambda b: (b, 0, 0)),
                pl.BlockSpec((1, 1, P, PW), lambda b, PW=PW: (0, b, 0, 0)),
                pl.BlockSpec((1, 1, P, PW), lambda b, PW=PW: (1, b, 0, 0)),
                pl.BlockSpec((1, P, 1), lambda b: (b, 0, 0)),
                pl.BlockSpec((1, C), lambda b: (0, 0)),
                pl.BlockSpec((9, C), lambda b: (0, 0)),
                pl.BlockSpec((1, C), lambda b: (0, 0)),
            ],
            out_specs=pl.BlockSpec((1, H, W, C), lambda b: (b, 0, 0, 0)),
            out_shape=jax.ShapeDtypeStruct((GB, H, W, C), jnp.float32),
        )(yg, gr, gr, w1, bf, taps, bdw))

    out = jnp.concatenate(outs, axis=0)
    return out.transpose(0, 3, 1, 2)


# speedup vs baseline: 1.0732x; 1.0592x over previous
import functools

import jax
import jax.numpy as jnp
from jax import lax
from jax.experimental import pallas as pl
from jax.experimental.pallas import tpu as pltpu
from jax.experimental.pallas import tpu_sc as plsc

_DIL = 2
_NEG_INF = float("-inf")

_SC_INFO = plsc.get_sparse_core_info()
_NC = _SC_INFO.num_cores
_NS = _SC_INFO.num_subcores
_NW = _NC * _NS
_CHUNK = 128


def _dot(a, b, ca, cb):
    return lax.dot_general(a, b, (((ca,), (cb,)), ((), ())),
                           preferred_element_type=jnp.float32)


def _score_body(x_ref, mem_ref, wf_ref, yg_ref, idx_ref, w1_ref, memB_ref):
    P = x_ref.shape[2]
    C = x_ref.shape[1]
    M = mem_ref.shape[0]
    mem = mem_ref[...]
    x = x_ref[0]
    xt = x.T

    @pl.when(pl.program_id(0) == 0)
    def _prep():
        C2 = C // 2
        mb = _dot(mem, wf_ref[:, C:], 1, 1).astype(jnp.bfloat16)
        lo = lax.bitcast_convert_type(mb[:, :C2], jnp.uint16).astype(jnp.uint32)
        hi = lax.bitcast_convert_type(mb[:, C2:], jnp.uint16).astype(jnp.uint32)
        packed = lo | (hi << 16)
        pad = memB_ref.shape[1] - C2
        memB_ref[...] = jnp.concatenate(
            [packed, jnp.zeros((M, pad), jnp.uint32)], axis=1)

    ig = jnp.mean(xt, axis=0, keepdims=True)
    sg = _dot(ig, mem, 1, 1)
    sg = sg - jnp.max(sg, axis=1, keepdims=True)
    eg = jnp.exp(sg)
    smg = eg / jnp.sum(eg, axis=1, keepdims=True)
    mr = jnp.dot(smg, mem, preferred_element_type=jnp.float32) + ig
    gate = 1.0 / (1.0 + jnp.exp(-mr))
    yg_ref[0] = _dot(xt * gate, wf_ref[:, :C], 1, 1)

    S = _dot(xt, mem, 1, 1)
    col = lax.broadcasted_iota(jnp.int32, (P, M), 1)
    v1 = jnp.max(S, axis=1, keepdims=True)
    i1 = jnp.min(jnp.where(S == v1, col, M), axis=1, keepdims=True)
    S2 = jnp.where(col == i1, _NEG_INF, S)
    v2 = jnp.max(S2, axis=1, keepdims=True)
    i2 = jnp.min(jnp.where(S2 == v2, col, M), axis=1, keepdims=True)
    e2 = jnp.exp(v2 - v1)
    idx_ref[0, 0] = i1
    idx_ref[1, 0] = i2
    w1_ref[0] = 1.0 / (1.0 + e2)


def _make_sc_gather(n_idx, D):
    rows_per_w = n_idx // _NW
    chunks = rows_per_w // _CHUNK
    mesh = plsc.VectorSubcoreMesh(core_axis_name="c", subcore_axis_name="s")

    @functools.partial(
        pl.kernel, mesh=mesh,
        out_type=jax.ShapeDtypeStruct((n_idx, D), jnp.uint32),
        scratch_types=[
            pltpu.VMEM((chunks, _CHUNK), jnp.int32),
            pltpu.VMEM((_CHUNK, D), jnp.uint32),
            pltpu.SemaphoreType.DMA,
        ],
    )
    def gather_k(table_hbm, idx_hbm, out_hbm, idx_v, rows_v, sem):
        wid = lax.axis_index("s") * _NC + lax.axis_index("c")
        base_chunk = wid * chunks
        pltpu.sync_copy(idx_hbm.at[pl.ds(base_chunk, chunks)], idx_v)
        for j in range(chunks):
            pltpu.async_copy(table_hbm.at[idx_v.at[j]], rows_v, sem).wait()
            pltpu.sync_copy(
                rows_v, out_hbm.at[pl.ds((base_chunk + j) * _CHUNK, _CHUNK)])

    return gather_k


def _fuse_body(H, W, yg_ref, g1_ref, g2_ref, w1_ref, bf_ref, taps_ref,
               bdw_ref, out_ref):
    C = yg_ref.shape[2]
    a1 = w1_ref[0]

    def unpack(gu_padded):
        gu = gu_padded[:, :C // 2]
        lo = lax.bitcast_convert_type(
            (gu & 0xFFFF).astype(jnp.uint16), jnp.bfloat16).astype(jnp.float32)
        hi = lax.bitcast_convert_type(
            (gu >> 16).astype(jnp.uint16), jnp.bfloat16).astype(jnp.float32)
        return jnp.concatenate([lo, hi], axis=1)

    g1 = unpack(g1_ref[0, 0])
    g2 = unpack(g2_ref[0, 0])
    Y = (yg_ref[0] + a1 * g1 + (1.0 - a1) * g2
         + bf_ref[...])
    Y = jnp.where(Y > 0, Y, 0.2 * Y)

    Yh = Y.reshape(H, W, C)

    def shift(a, axis, d):
        if d == 0:
            return a
        zshape = list(a.shape)
        zshape[axis] = abs(d)
        z = jnp.zeros(zshape, a.dtype)
        n = a.shape[axis]
        if d > 0:
            body = lax.slice_in_dim(a, d, n, axis=axis)
            return jnp.concatenate([body, z], axis=axis)
        body = lax.slice_in_dim(a, 0, n + d, axis=axis)
        return jnp.concatenate([z, body], axis=axis)

    acc = jnp.zeros((H, W, C), jnp.float32)
    k = 0
    for kh in range(3):
        for kw in range(3):
            dh = (kh - 1) * _DIL
            dw = (kw - 1) * _DIL
            win = shift(shift(Yh, 0, dh), 1, dw)
            acc = acc + win * taps_ref[k, :][None, None, :]
            k += 1
    acc = acc + bdw_ref[0, :][None, None, :]
    out_ref[0] = jnp.where(acc > 0, acc, 0.2 * acc)


def kernel(image_feature, memory, W_fuse, b_fuse, W_dw, b_dw):
    B, C, H, W = image_feature.shape
    M = memory.shape[0]
    P = H * W
    GB = 4
    x_cp = image_feature.reshape(B, C, P)
    taps = W_dw[:, 0, :, :].reshape(C, 9).T
    bf = b_fuse.reshape(1, C)
    bdw = b_dw.reshape(1, C)

    PW = 256
    sc_gather = _make_sc_gather(2 * GB * P, PW)
    outs = []
    for g in range(B // GB):
        yg, idx, w1, memB = pl.pallas_call(
            _score_body,
            grid=(GB,),
            in_specs=[
                pl.BlockSpec((1, C, P), lambda b, g=g: (GB * g + b, 0, 0)),
                pl.BlockSpec((M, C), lambda b: (0, 0)),
                pl.BlockSpec((C, 2 * C), lambda b: (0, 0)),
            ],
            out_specs=[
                pl.BlockSpec((1, P, C), lambda b: (b, 0, 0)),
                pl.BlockSpec((2, 1, P, 1), lambda b: (0, b, 0, 0)),
                pl.BlockSpec((1, P, 1), lambda b: (b, 0, 0)),
                pl.BlockSpec((M, PW), lambda b: (0, 0)),
            ],
            out_shape=[
                jax.ShapeDtypeStruct((GB, P, C), jnp.float32),
                jax.ShapeDtypeStruct((2, GB, P, 1), jnp.int32),
                jax.ShapeDtypeStruct((GB, P, 1), jnp.float32),
                jax.ShapeDtypeStruct((M, PW), jnp.uint32),
            ],
        )(x_cp, memory, W_fuse)

        n_idx = 2 * GB * P
        idx_flat = idx.reshape(n_idx // _CHUNK, _CHUNK)
        gathered = sc_gather(memB, idx_flat)
        gr = gathered.reshape(2, GB, P, PW)

        outs.append(pl.pallas_call(
            functools.partial(_fuse_body, H, W),
            grid=(GB,),
            in_specs=[
                pl.BlockSpec((1, P, C), lambda b: (b, 0, 0)),
                pl.BlockSpec((1, 1, P, PW), lambda b, PW=PW: (0, b, 0, 0)),
                pl.BlockSpec((1, 1, P, PW), lambda b, PW=PW: (1, b, 0, 0)),
                pl.BlockSpec((1, P, 1), lambda b: (b, 0, 0)),
                pl.BlockSpec((1, C), lambda b: (0, 0)),
                pl.BlockSpec((9, C), lambda b: (0, 0)),
                pl.BlockSpec((1, C), lambda b: (0, 0)),
            ],
            out_specs=pl.BlockSpec((1, H, W, C), lambda b: (b, 0, 0, 0)),
            out_shape=jax.ShapeDtypeStruct((GB, H, W, C), jnp.float32),
        )(yg, gr, gr, w1, bf, taps, bdw))

    out = jnp.concatenate(outs, axis=0)
    return out.transpose(0, 3, 1, 2)


# speedup vs baseline: 1.1183x; 1.0420x over previous
import functools

import jax
import jax.numpy as jnp
from jax import lax
from jax.experimental import pallas as pl
from jax.experimental.pallas import tpu as pltpu
from jax.experimental.pallas import tpu_sc as plsc

_DIL = 2
_NEG_INF = float("-inf")

_SC_INFO = plsc.get_sparse_core_info()
_NC = _SC_INFO.num_cores
_NS = _SC_INFO.num_subcores
_NW = _NC * _NS
_CHUNK = 128


def _dot(a, b, ca, cb):
    return lax.dot_general(a, b, (((ca,), (cb,)), ((), ())),
                           preferred_element_type=jnp.float32)


def _score_body(x_ref, mem_ref, wf_ref, yg_ref, idx_ref, w1_ref, memB_ref):
    P = x_ref.shape[2]
    C = x_ref.shape[1]
    M = mem_ref.shape[0]
    mem = mem_ref[...]
    x = x_ref[0]
    xt = x.T

    @pl.when(pl.program_id(0) == 0)
    def _prep():
        C2 = C // 2
        mb = _dot(mem, wf_ref[:, C:], 1, 1).astype(jnp.bfloat16)
        lo = lax.bitcast_convert_type(mb[:, :C2], jnp.uint16).astype(jnp.uint32)
        hi = lax.bitcast_convert_type(mb[:, C2:], jnp.uint16).astype(jnp.uint32)
        packed = lo | (hi << 16)
        pad = memB_ref.shape[1] - C2
        memB_ref[...] = jnp.concatenate(
            [packed, jnp.zeros((M, pad), jnp.uint32)], axis=1)

    ig = jnp.mean(xt, axis=0, keepdims=True)
    sg = _dot(ig, mem, 1, 1)
    sg = sg - jnp.max(sg, axis=1, keepdims=True)
    eg = jnp.exp(sg)
    smg = eg / jnp.sum(eg, axis=1, keepdims=True)
    mr = jnp.dot(smg, mem, preferred_element_type=jnp.float32) + ig
    gate = 1.0 / (1.0 + jnp.exp(-mr))
    yg_ref[0] = _dot(xt * gate, wf_ref[:, :C], 1, 1)

    S = _dot(xt, mem, 1, 1)
    col = lax.broadcasted_iota(jnp.int32, (P, M), 1)
    v1 = jnp.max(S, axis=1, keepdims=True)
    i1 = jnp.min(jnp.where(S == v1, col, M), axis=1, keepdims=True)
    S2 = jnp.where(col == i1, _NEG_INF, S)
    v2 = jnp.max(S2, axis=1, keepdims=True)
    i2 = jnp.min(jnp.where(S2 == v2, col, M), axis=1, keepdims=True)
    e2 = jnp.exp(v2 - v1)
    idx_ref[0, 0] = i1.T
    idx_ref[1, 0] = i2.T
    w1_ref[0] = 1.0 / (1.0 + e2)


def _make_sc_gather(n_idx, D):
    rows_per_w = n_idx // _NW
    chunks = rows_per_w // _CHUNK
    mesh = plsc.VectorSubcoreMesh(core_axis_name="c", subcore_axis_name="s")

    @functools.partial(
        pl.kernel, mesh=mesh,
        out_type=jax.ShapeDtypeStruct((n_idx, D), jnp.uint32),
        scratch_types=[
            pltpu.VMEM((chunks, _CHUNK), jnp.int32),
            pltpu.VMEM((_CHUNK, D), jnp.uint32),
            pltpu.SemaphoreType.DMA,
        ],
    )
    def gather_k(table_hbm, idx_hbm, out_hbm, idx_v, rows_v, sem):
        wid = lax.axis_index("s") * _NC + lax.axis_index("c")
        base_chunk = wid * chunks
        pltpu.sync_copy(idx_hbm.at[pl.ds(base_chunk, chunks)], idx_v)
        for j in range(chunks):
            pltpu.async_copy(table_hbm.at[idx_v.at[j]], rows_v, sem).wait()
            pltpu.sync_copy(
                rows_v, out_hbm.at[pl.ds((base_chunk + j) * _CHUNK, _CHUNK)])

    return gather_k


def _fuse_body(H, W, yg_ref, g1_ref, g2_ref, w1_ref, bf_ref, taps_ref,
               bdw_ref, out_ref):
    C = yg_ref.shape[2]
    a1 = w1_ref[0]

    def unpack(gu_padded):
        gu = gu_padded[:, :C // 2]
        lo = lax.bitcast_convert_type(
            (gu & 0xFFFF).astype(jnp.uint16), jnp.bfloat16).astype(jnp.float32)
        hi = lax.bitcast_convert_type(
            (gu >> 16).astype(jnp.uint16), jnp.bfloat16).astype(jnp.float32)
        return jnp.concatenate([lo, hi], axis=1)

    g1 = unpack(g1_ref[0, 0])
    g2 = unpack(g2_ref[0, 0])
    Y = (yg_ref[0] + a1 * g1 + (1.0 - a1) * g2
         + bf_ref[...])
    Y = jnp.where(Y > 0, Y, 0.2 * Y)

    Yh = Y.reshape(H, W, C)

    def shift(a, axis, d):
        if d == 0:
            return a
        zshape = list(a.shape)
        zshape[axis] = abs(d)
        z = jnp.zeros(zshape, a.dtype)
        n = a.shape[axis]
        if d > 0:
            body = lax.slice_in_dim(a, d, n, axis=axis)
            return jnp.concatenate([body, z], axis=axis)
        body = lax.slice_in_dim(a, 0, n + d, axis=axis)
        return jnp.concatenate([z, body], axis=axis)

    acc = jnp.zeros((H, W, C), jnp.float32)
    k = 0
    for kh in range(3):
        for kw in range(3):
            dh = (kh - 1) * _DIL
            dw = (kw - 1) * _DIL
            win = shift(shift(Yh, 0, dh), 1, dw)
            acc = acc + win * taps_ref[k, :][None, None, :]
            k += 1
    acc = acc + bdw_ref[0, :][None, None, :]
    out_ref[0] = jnp.where(acc > 0, acc, 0.2 * acc)


def kernel(image_feature, memory, W_fuse, b_fuse, W_dw, b_dw):
    B, C, H, W = image_feature.shape
    M = memory.shape[0]
    P = H * W
    GB = 4
    x_cp = image_feature.reshape(B, C, P)
    taps = W_dw[:, 0, :, :].reshape(C, 9).T
    bf = b_fuse.reshape(1, C)
    bdw = b_dw.reshape(1, C)

    PW = 256
    sc_gather = _make_sc_gather(2 * GB * P, PW)
    outs = []
    for g in range(B // GB):
        yg, idx, w1, memB = pl.pallas_call(
            _score_body,
            grid=(GB,),
            in_specs=[
                pl.BlockSpec((1, C, P), lambda b, g=g: (GB * g + b, 0, 0)),
                pl.BlockSpec((M, C), lambda b: (0, 0)),
                pl.BlockSpec((C, 2 * C), lambda b: (0, 0)),
            ],
            out_specs=[
                pl.BlockSpec((1, P, C), lambda b: (b, 0, 0)),
                pl.BlockSpec((2, 1, 1, P), lambda b: (0, b, 0, 0)),
                pl.BlockSpec((1, P, 1), lambda b: (b, 0, 0)),
                pl.BlockSpec((M, PW), lambda b: (0, 0)),
            ],
            out_shape=[
                jax.ShapeDtypeStruct((GB, P, C), jnp.float32),
                jax.ShapeDtypeStruct((2, GB, 1, P), jnp.int32),
                jax.ShapeDtypeStruct((GB, P, 1), jnp.float32),
                jax.ShapeDtypeStruct((M, PW), jnp.uint32),
            ],
        )(x_cp, memory, W_fuse)

        n_idx = 2 * GB * P
        idx_flat = idx.reshape(n_idx // _CHUNK, _CHUNK)
        gathered = sc_gather(memB, idx_flat)
        gr = gathered.reshape(2, GB, P, PW)

        outs.append(pl.pallas_call(
            functools.partial(_fuse_body, H, W),
            grid=(GB,),
            in_specs=[
                pl.BlockSpec((1, P, C), lambda b: (b, 0, 0)),
                pl.BlockSpec((1, 1, P, PW), lambda b, PW=PW: (0, b, 0, 0)),
                pl.BlockSpec((1, 1, P, PW), lambda b, PW=PW: (1, b, 0, 0)),
                pl.BlockSpec((1, P, 1), lambda b: (b, 0, 0)),
                pl.BlockSpec((1, C), lambda b: (0, 0)),
                pl.BlockSpec((9, C), lambda b: (0, 0)),
                pl.BlockSpec((1, C), lambda b: (0, 0)),
            ],
            out_specs=pl.BlockSpec((1, H, W, C), lambda b: (b, 0, 0, 0)),
            out_shape=jax.ShapeDtypeStruct((GB, H, W, C), jnp.float32),
        )(yg, gr, gr, w1, bf, taps, bdw))

    out = jnp.concatenate(outs, axis=0)
    return out.transpose(0, 3, 1, 2)


# speedup vs baseline: 1.1271x; 1.0079x over previous
import functools

import jax
import jax.numpy as jnp
from jax import lax
from jax.experimental import pallas as pl
from jax.experimental.pallas import tpu as pltpu
from jax.experimental.pallas import tpu_sc as plsc

_DIL = 2
_NEG_INF = float("-inf")

_SC_INFO = plsc.get_sparse_core_info()
_NC = _SC_INFO.num_cores
_NS = _SC_INFO.num_subcores
_NW = _NC * _NS
_CHUNK = 128


def _dot(a, b, ca, cb):
    return lax.dot_general(a, b, (((ca,), (cb,)), ((), ())),
                           preferred_element_type=jnp.float32)


def _score_body(x_ref, mem_ref, wf_ref, yg_ref, idx_ref, w1_ref, memB_ref):
    P = x_ref.shape[2]
    C = x_ref.shape[1]
    M = mem_ref.shape[0]
    mem = mem_ref[...]
    x = x_ref[0]
    xt = x.T

    @pl.when(pl.program_id(0) == 0)
    def _prep():
        C2 = C // 2
        mb = _dot(mem, wf_ref[:, C:], 1, 1).astype(jnp.bfloat16)
        lo = lax.bitcast_convert_type(mb[:, :C2], jnp.uint16).astype(jnp.uint32)
        hi = lax.bitcast_convert_type(mb[:, C2:], jnp.uint16).astype(jnp.uint32)
        packed = lo | (hi << 16)
        pad = memB_ref.shape[1] - C2
        memB_ref[...] = jnp.concatenate(
            [packed, jnp.zeros((M, pad), jnp.uint32)], axis=1)

    ig = jnp.mean(xt, axis=0, keepdims=True)
    sg = _dot(ig, mem, 1, 1)
    sg = sg - jnp.max(sg, axis=1, keepdims=True)
    eg = jnp.exp(sg)
    smg = eg / jnp.sum(eg, axis=1, keepdims=True)
    mr = jnp.dot(smg, mem, preferred_element_type=jnp.float32) + ig
    gate = 1.0 / (1.0 + jnp.exp(-mr))
    yg_ref[0] = _dot(xt * gate, wf_ref[:, :C], 1, 1)

    S = _dot(xt, mem, 1, 1)
    col = lax.broadcasted_iota(jnp.int32, (P, M), 1)
    v1 = jnp.max(S, axis=1, keepdims=True)
    i1 = jnp.min(jnp.where(S == v1, col, M), axis=1, keepdims=True)
    S2 = jnp.where(col == i1, _NEG_INF, S)
    v2 = jnp.max(S2, axis=1, keepdims=True)
    i2 = jnp.min(jnp.where(S2 == v2, col, M), axis=1, keepdims=True)
    e2 = jnp.exp(v2 - v1)
    idx_ref[0, 0] = i1.T
    idx_ref[1, 0] = i2.T
    w1_ref[0] = (1.0 / (1.0 + e2)).T


def _make_sc_gather(n_idx, D):
    rows_per_w = n_idx // _NW
    chunks = rows_per_w // _CHUNK
    mesh = plsc.VectorSubcoreMesh(core_axis_name="c", subcore_axis_name="s")

    @functools.partial(
        pl.kernel, mesh=mesh,
        out_type=jax.ShapeDtypeStruct((n_idx, D), jnp.uint32),
        scratch_types=[
            pltpu.VMEM((chunks, _CHUNK), jnp.int32),
            pltpu.VMEM((_CHUNK, D), jnp.uint32),
            pltpu.SemaphoreType.DMA,
        ],
    )
    def gather_k(table_hbm, idx_hbm, out_hbm, idx_v, rows_v, sem):
        wid = lax.axis_index("s") * _NC + lax.axis_index("c")
        base_chunk = wid * chunks
        pltpu.sync_copy(idx_hbm.at[pl.ds(base_chunk, chunks)], idx_v)
        for j in range(chunks):
            pltpu.async_copy(table_hbm.at[idx_v.at[j]], rows_v, sem).wait()
            pltpu.sync_copy(
                rows_v, out_hbm.at[pl.ds((base_chunk + j) * _CHUNK, _CHUNK)])

    return gather_k


def _fuse_body(H, W, yg_ref, g1_ref, g2_ref, w1_ref, bf_ref, taps_ref,
               bdw_ref, out_ref):
    C = yg_ref.shape[2]
    a1 = w1_ref[0].T

    def unpack(gu_padded):
        gu = gu_padded[:, :C // 2]
        lo = lax.bitcast_convert_type(
            (gu & 0xFFFF).astype(jnp.uint16), jnp.bfloat16).astype(jnp.float32)
        hi = lax.bitcast_convert_type(
            (gu >> 16).astype(jnp.uint16), jnp.bfloat16).astype(jnp.float32)
        return jnp.concatenate([lo, hi], axis=1)

    g1 = unpack(g1_ref[0, 0])
    g2 = unpack(g2_ref[0, 0])
    Y = (yg_ref[0] + a1 * g1 + (1.0 - a1) * g2
         + bf_ref[...])
    Y = jnp.where(Y > 0, Y, 0.2 * Y)

    Yh = Y.reshape(H, W, C)

    def shift(a, axis, d):
        if d == 0:
            return a
        zshape = list(a.shape)
        zshape[axis] = abs(d)
        z = jnp.zeros(zshape, a.dtype)
        n = a.shape[axis]
        if d > 0:
            body = lax.slice_in_dim(a, d, n, axis=axis)
            return jnp.concatenate([body, z], axis=axis)
        body = lax.slice_in_dim(a, 0, n + d, axis=axis)
        return jnp.concatenate([z, body], axis=axis)

    acc = jnp.zeros((H, W, C), jnp.float32)
    k = 0
    for kh in range(3):
        for kw in range(3):
            dh = (kh - 1) * _DIL
            dw = (kw - 1) * _DIL
            win = shift(shift(Yh, 0, dh), 1, dw)
            acc = acc + win * taps_ref[k, :][None, None, :]
            k += 1
    acc = acc + bdw_ref[0, :][None, None, :]
    out_ref[0] = jnp.where(acc > 0, acc, 0.2 * acc)


def kernel(image_feature, memory, W_fuse, b_fuse, W_dw, b_dw):
    B, C, H, W = image_feature.shape
    M = memory.shape[0]
    P = H * W
    GB = 4
    x_cp = image_feature.reshape(B, C, P)
    taps = W_dw[:, 0, :, :].reshape(C, 9).T
    bf = b_fuse.reshape(1, C)
    bdw = b_dw.reshape(1, C)

    PW = 256
    sc_gather = _make_sc_gather(2 * GB * P, PW)
    outs = []
    for g in range(B // GB):
        yg, idx, w1, memB = pl.pallas_call(
            _score_body,
            grid=(GB,),
            in_specs=[
                pl.BlockSpec((1, C, P), lambda b, g=g: (GB * g + b, 0, 0)),
                pl.BlockSpec((M, C), lambda b: (0, 0)),
                pl.BlockSpec((C, 2 * C), lambda b: (0, 0)),
            ],
            out_specs=[
                pl.BlockSpec((1, P, C), lambda b: (b, 0, 0)),
                pl.BlockSpec((2, 1, 1, P), lambda b: (0, b, 0, 0)),
                pl.BlockSpec((1, 1, P), lambda b: (b, 0, 0)),
                pl.BlockSpec((M, PW), lambda b: (0, 0)),
            ],
            out_shape=[
                jax.ShapeDtypeStruct((GB, P, C), jnp.float32),
                jax.ShapeDtypeStruct((2, GB, 1, P), jnp.int32),
                jax.ShapeDtypeStruct((GB, 1, P), jnp.float32),
                jax.ShapeDtypeStruct((M, PW), jnp.uint32),
            ],
        )(x_cp, memory, W_fuse)

        n_idx = 2 * GB * P
        idx_flat = idx.reshape(n_idx // _CHUNK, _CHUNK)
        gathered = sc_gather(memB, idx_flat)
        gr = gathered.reshape(2, GB, P, PW)

        outs.append(pl.pallas_call(
            functools.partial(_fuse_body, H, W),
            grid=(GB,),
            in_specs=[
                pl.BlockSpec((1, P, C), lambda b: (b, 0, 0)),
                pl.BlockSpec((1, 1, P, PW), lambda b, PW=PW: (0, b, 0, 0)),
                pl.BlockSpec((1, 1, P, PW), lambda b, PW=PW: (1, b, 0, 0)),
                pl.BlockSpec((1, 1, P), lambda b: (b, 0, 0)),
                pl.BlockSpec((1, C), lambda b: (0, 0)),
                pl.BlockSpec((9, C), lambda b: (0, 0)),
                pl.BlockSpec((1, C), lambda b: (0, 0)),
            ],
            out_specs=pl.BlockSpec((1, H, W, C), lambda b: (b, 0, 0, 0)),
            out_shape=jax.ShapeDtypeStruct((GB, H, W, C), jnp.float32),
        )(yg, gr, gr, w1, bf, taps, bdw))

    out = jnp.concatenate(outs, axis=0)
    return out.transpose(0, 3, 1, 2)
